# Initial kernel scaffold; baseline (speedup 1.0000x reference)
import jax
import jax.numpy as jnp
from jax.experimental import pallas as pl


def kernel(uid, iid, uid_table, iid_table, lr_w, lr_b):
    raise NotImplementedError("write your pallas kernel here")


# speedup vs baseline: 2.9919x; 2.9919x over previous
import functools

import jax
import jax.numpy as jnp
from jax import lax
from jax.experimental import pallas as pl
from jax.experimental.pallas import tpu as pltpu
from jax.experimental.pallas import tpu_sc as plsc

_EMB = 32
_BATCH = 16384
_NC = 2
_NS = 16
_NW = _NC * _NS
_BPW = _BATCH // _NW
_RING = 4

_GDN = lax.GatherDimensionNumbers(
    offset_dims=(), collapsed_slice_dims=(0,), start_index_map=(0,))


def _lane_splat(x, lane):
    idx = jnp.full((16,), 0, jnp.int32) + lane
    return lax.gather(x, idx[:, None], _GDN, slice_sizes=(1,),
                      mode=lax.GatherScatterMode.PROMISE_IN_BOUNDS)


def _sc_body(uid_hbm, iid_hbm, ut_hbm, it_hbm, wb_hbm, out_hbm,
             idx_u, idx_i, slots_u, slots_i, w_v, out_v, sem_u, sem_i):
    wid = lax.axis_index("c") * _NS + lax.axis_index("s")
    base = wid * _BPW

    pltpu.sync_copy(uid_hbm.at[pl.ds(base, _BPW)], idx_u)
    pltpu.sync_copy(iid_hbm.at[pl.ds(base, _BPW)], idx_i)
    pltpu.sync_copy(wb_hbm, w_v)

    def fetch(tab, slots, sem, u, r):
        blk = pl.multiple_of((u // 128) * 128, 128)
        pltpu.async_copy(tab.at[:, pl.ds(blk, 128)],
                         slots.at[pl.ds(r * _EMB, _EMB), :], sem)

    def drain(tab, slots, sem):
        pltpu.make_async_copy(tab.at[:, pl.ds(0, 128)],
                              slots.at[pl.ds(0, _EMB), :], sem).wait()

    wreg = [w_v[pl.ds(k * 16, 16)] for k in range(5)]
    bias = wreg[4][0]
    iota16 = lax.iota(jnp.int32, 16)

    def colsum(slots, r, u, woff):
        c16 = pl.multiple_of(((u % 128) // 16) * 16, 16)
        acc = jnp.zeros((16,), jnp.float32)
        for j in range(_EMB):
            w = wreg[woff + j // 16][j % 16]
            acc = acc + slots[r * _EMB + j, pl.ds(c16, 16)] * w
        return _lane_splat(acc, u % 16)

    iv_u0 = idx_u[pl.ds(0, 16)]
    iv_i0 = idx_i[pl.ds(0, 16)]
    for r in range(_RING):
        fetch(ut_hbm, slots_u, sem_u, iv_u0[r], r)
        fetch(it_hbm, slots_i, sem_i, iv_i0[r], r)

    def group(g, carry):
        iv_u = idx_u[pl.ds(g * 16, 16)]
        iv_i = idx_i[pl.ds(g * 16, 16)]
        gn = jnp.where(g < (_BPW // 16) - 1, g + 1, g)
        ivn_u = idx_u[pl.ds(gn * 16, 16)]
        ivn_i = idx_i[pl.ds(gn * 16, 16)]
        acc = jnp.zeros((16,), jnp.float32)
        for k in range(16):
            r = k % _RING
            drain(ut_hbm, slots_u, sem_u)
            drain(it_hbm, slots_i, sem_i)
            su = colsum(slots_u, r, iv_u[k], 0)
            si = colsum(slots_i, r, iv_i[k], 2)
            acc = jnp.where(iota16 == k, su + si, acc)
            if k < 16 - _RING:
                un, vn = iv_u[k + _RING], iv_i[k + _RING]
            else:
                un, vn = ivn_u[k + _RING - 16], ivn_i[k + _RING - 16]
            fetch(ut_hbm, slots_u, sem_u, un, r)
            fetch(it_hbm, slots_i, sem_i, vn, r)
        x = acc + bias
        y = 1.0 / (1.0 + jnp.exp(-x))
        out_v[pl.ds(g * 16, 16)] = y
        return carry

    lax.fori_loop(0, _BPW // 16, group, 0)
    for _ in range(_RING):
        drain(ut_hbm, slots_u, sem_u)
        drain(it_hbm, slots_i, sem_i)
    pltpu.sync_copy(out_v, out_hbm.at[pl.ds(base, _BPW)])


@functools.partial(jax.jit)
def _lrrank_sc(uid, iid, ut_t, it_t, wb):
    mesh = plsc.VectorSubcoreMesh(core_axis_name="c", subcore_axis_name="s")
    fn = pl.kernel(
        _sc_body,
        out_type=jax.ShapeDtypeStruct((_BATCH,), jnp.float32),
        mesh=mesh,
        scratch_types=[
            pltpu.VMEM((_BPW,), jnp.int32),
            pltpu.VMEM((_BPW,), jnp.int32),
            pltpu.VMEM((_RING * _EMB, 128), jnp.float32),
            pltpu.VMEM((_RING * _EMB, 128), jnp.float32),
            pltpu.VMEM((80,), jnp.float32),
            pltpu.VMEM((_BPW,), jnp.float32),
            pltpu.SemaphoreType.DMA,
            pltpu.SemaphoreType.DMA,
        ],
    )
    return fn(uid, iid, ut_t, it_t, wb)


def kernel(uid, iid, uid_table, iid_table, lr_w, lr_b):
    wb = jnp.concatenate(
        [lr_w.reshape(-1), lr_b.reshape(-1),
         jnp.zeros((15,), jnp.float32)])
    y = _lrrank_sc(uid.astype(jnp.int32), iid.astype(jnp.int32),
                   uid_table.T, iid_table.T, wb)
    return y.reshape(_BATCH, 1)


# speedup vs baseline: 3.1770x; 1.0619x over previous
import functools

import jax
import jax.numpy as jnp
from jax import lax
from jax.experimental import pallas as pl
from jax.experimental.pallas import tpu as pltpu
from jax.experimental.pallas import tpu_sc as plsc

_EMB = 32
_BATCH = 16384
_NC = 2
_NS = 16
_NW = _NC * _NS
_BPW = _BATCH // _NW
_RING = 8

_GDN = lax.GatherDimensionNumbers(
    offset_dims=(), collapsed_slice_dims=(0,), start_index_map=(0,))


def _lane_splat(x, lane):
    idx = jnp.full((16,), 0, jnp.int32) + lane
    return lax.gather(x, idx[:, None], _GDN, slice_sizes=(1,),
                      mode=lax.GatherScatterMode.PROMISE_IN_BOUNDS)


def _sc_body(uid_hbm, iid_hbm, ut_hbm, it_hbm, wb_hbm, out_hbm,
             idx_u, idx_i, slots_u, slots_i, w_v, out_v, sem_u, sem_i):
    wid = lax.axis_index("c") * _NS + lax.axis_index("s")
    base = wid * _BPW

    pltpu.sync_copy(uid_hbm.at[pl.ds(base, _BPW)], idx_u)
    pltpu.sync_copy(iid_hbm.at[pl.ds(base, _BPW)], idx_i)
    pltpu.sync_copy(wb_hbm, w_v)

    def fetch(tab, slots, sem, u, r):
        blk = pl.multiple_of((u // 128) * 128, 128)
        pltpu.async_copy(tab.at[:, pl.ds(blk, 128)],
                         slots.at[pl.ds(r * _EMB, _EMB), :], sem)

    def drain(tab, slots, sem):
        pltpu.make_async_copy(tab.at[:, pl.ds(0, 128)],
                              slots.at[pl.ds(0, _EMB), :], sem).wait()

    wreg = [w_v[pl.ds(k * 16, 16)] for k in range(5)]
    bias = wreg[4][0]
    iota16 = lax.iota(jnp.int32, 16)

    def colsum(slots, r, u, woff):
        c16 = pl.multiple_of(((u % 128) // 16) * 16, 16)
        acc = jnp.zeros((16,), jnp.float32)
        for j in range(_EMB):
            w = wreg[woff + j // 16][j % 16]
            acc = acc + slots[r * _EMB + j, pl.ds(c16, 16)] * w
        return _lane_splat(acc, u % 16)

    iv_u0 = idx_u[pl.ds(0, 16)]
    iv_i0 = idx_i[pl.ds(0, 16)]
    for r in range(_RING):
        fetch(ut_hbm, slots_u, sem_u, iv_u0[r], r)
        fetch(it_hbm, slots_i, sem_i, iv_i0[r], r)

    def group(g, carry):
        iv_u = idx_u[pl.ds(g * 16, 16)]
        iv_i = idx_i[pl.ds(g * 16, 16)]
        gn = jnp.where(g < (_BPW // 16) - 1, g + 1, g)
        ivn_u = idx_u[pl.ds(gn * 16, 16)]
        ivn_i = idx_i[pl.ds(gn * 16, 16)]
        acc = jnp.zeros((16,), jnp.float32)
        for k in range(16):
            r = k % _RING
            drain(ut_hbm, slots_u, sem_u)
            drain(it_hbm, slots_i, sem_i)
            su = colsum(slots_u, r, iv_u[k], 0)
            si = colsum(slots_i, r, iv_i[k], 2)
            acc = jnp.where(iota16 == k, su + si, acc)
            if k < 16 - _RING:
                un, vn = iv_u[k + _RING], iv_i[k + _RING]
            else:
                un, vn = ivn_u[k + _RING - 16], ivn_i[k + _RING - 16]
            fetch(ut_hbm, slots_u, sem_u, un, r)
            fetch(it_hbm, slots_i, sem_i, vn, r)
        x = acc + bias
        y = 1.0 / (1.0 + jnp.exp(-x))
        out_v[pl.ds(g * 16, 16)] = y
        return carry

    lax.fori_loop(0, _BPW // 16, group, 0)
    for _ in range(_RING):
        drain(ut_hbm, slots_u, sem_u)
        drain(it_hbm, slots_i, sem_i)
    pltpu.sync_copy(out_v, out_hbm.at[pl.ds(base, _BPW)])


@functools.partial(jax.jit)
def _lrrank_sc(uid, iid, ut_t, it_t, wb):
    mesh = plsc.VectorSubcoreMesh(core_axis_name="c", subcore_axis_name="s")
    fn = pl.kernel(
        _sc_body,
        out_type=jax.ShapeDtypeStruct((_BATCH,), jnp.float32),
        mesh=mesh,
        scratch_types=[
            pltpu.VMEM((_BPW,), jnp.int32),
            pltpu.VMEM((_BPW,), jnp.int32),
            pltpu.VMEM((_RING * _EMB, 128), jnp.float32),
            pltpu.VMEM((_RING * _EMB, 128), jnp.float32),
            pltpu.VMEM((80,), jnp.float32),
            pltpu.VMEM((_BPW,), jnp.float32),
            pltpu.SemaphoreType.DMA,
            pltpu.SemaphoreType.DMA,
        ],
    )
    return fn(uid, iid, ut_t, it_t, wb)


def kernel(uid, iid, uid_table, iid_table, lr_w, lr_b):
    wb = jnp.concatenate(
        [lr_w.reshape(-1), lr_b.reshape(-1),
         jnp.zeros((15,), jnp.float32)])
    y = _lrrank_sc(uid.astype(jnp.int32), iid.astype(jnp.int32),
                   uid_table.T, iid_table.T, wb)
    return y.reshape(_BATCH, 1)


# speedup vs baseline: 3.3779x; 1.0632x over previous
import functools

import jax
import jax.numpy as jnp
from jax import lax
from jax.experimental import pallas as pl
from jax.experimental.pallas import tpu as pltpu
from jax.experimental.pallas import tpu_sc as plsc

_EMB = 32
_BATCH = 16384
_NC = 2
_NS = 16
_NW = _NC * _NS
_BPW = _BATCH // _NW
_RING = 12

_GDN = lax.GatherDimensionNumbers(
    offset_dims=(), collapsed_slice_dims=(0,), start_index_map=(0,))


def _lane_splat(x, lane):
    idx = jnp.full((16,), 0, jnp.int32) + lane
    return lax.gather(x, idx[:, None], _GDN, slice_sizes=(1,),
                      mode=lax.GatherScatterMode.PROMISE_IN_BOUNDS)


def _sc_body(uid_hbm, iid_hbm, ut_hbm, it_hbm, wb_hbm, out_hbm,
             idx_u, idx_i, slots, w_v, out_v, sem):
    wid = lax.axis_index("c") * _NS + lax.axis_index("s")
    base = wid * _BPW

    pltpu.sync_copy(uid_hbm.at[pl.ds(base, _BPW)], idx_u)
    pltpu.sync_copy(iid_hbm.at[pl.ds(base, _BPW)], idx_i)
    pltpu.sync_copy(wb_hbm, w_v)

    def fetch(tab, u, r, half):
        blk = pl.multiple_of((u // 128) * 128, 128)
        pltpu.async_copy(
            tab.at[:, pl.ds(blk, 128)],
            slots.at[pl.ds(r * _EMB, _EMB), pl.ds(half * 128, 128)], sem)

    def drain2():
        pltpu.make_async_copy(ut_hbm.at[:, pl.ds(0, 256)],
                              slots.at[pl.ds(0, _EMB), :], sem).wait()

    wreg = [w_v[pl.ds(k * 16, 16)] for k in range(5)]
    bias = wreg[4][0]
    iota16 = lax.iota(jnp.int32, 16)

    def colsum(r, u, woff, half):
        c16 = pl.multiple_of(half * 128 + ((u % 128) // 16) * 16, 16)
        acc = jnp.zeros((16,), jnp.float32)
        for j in range(_EMB):
            w = wreg[woff + j // 16][j % 16]
            acc = acc + slots[r * _EMB + j, pl.ds(c16, 16)] * w
        return _lane_splat(acc, u % 16)

    iv_u0 = idx_u[pl.ds(0, 16)]
    iv_i0 = idx_i[pl.ds(0, 16)]
    for r in range(_RING):
        fetch(ut_hbm, iv_u0[r], r, 0)
        fetch(it_hbm, iv_i0[r], r, 1)

    def group(g, carry):
        iv_u = idx_u[pl.ds(g * 16, 16)]
        iv_i = idx_i[pl.ds(g * 16, 16)]
        gn = jnp.where(g < (_BPW // 16) - 1, g + 1, g)
        ivn_u = idx_u[pl.ds(gn * 16, 16)]
        ivn_i = idx_i[pl.ds(gn * 16, 16)]
        acc = jnp.zeros((16,), jnp.float32)
        for k in range(16):
            r = k % _RING
            drain2()
            su = colsum(r, iv_u[k], 0, 0)
            si = colsum(r, iv_i[k], 2, 1)
            acc = jnp.where(iota16 == k, su + si, acc)
            if k < 16 - _RING:
                un, vn = iv_u[k + _RING], iv_i[k + _RING]
            else:
                un, vn = ivn_u[k + _RING - 16], ivn_i[k + _RING - 16]
            fetch(ut_hbm, un, r, 0)
            fetch(it_hbm, vn, r, 1)
        x = acc + bias
        y = 1.0 / (1.0 + jnp.exp(-x))
        out_v[pl.ds(g * 16, 16)] = y
        return carry

    lax.fori_loop(0, _BPW // 16, group, 0)
    for _ in range(_RING):
        drain2()
    pltpu.sync_copy(out_v, out_hbm.at[pl.ds(base, _BPW)])


@functools.partial(jax.jit)
def _lrrank_sc(uid, iid, ut_t, it_t, wb):
    mesh = plsc.VectorSubcoreMesh(core_axis_name="c", subcore_axis_name="s")
    fn = pl.kernel(
        _sc_body,
        out_type=jax.ShapeDtypeStruct((_BATCH,), jnp.float32),
        mesh=mesh,
        scratch_types=[
            pltpu.VMEM((_BPW,), jnp.int32),
            pltpu.VMEM((_BPW,), jnp.int32),
            pltpu.VMEM((_RING * _EMB, 256), jnp.float32),
            pltpu.VMEM((80,), jnp.float32),
            pltpu.VMEM((_BPW,), jnp.float32),
            pltpu.SemaphoreType.DMA,
        ],
    )
    return fn(uid, iid, ut_t, it_t, wb)


def kernel(uid, iid, uid_table, iid_table, lr_w, lr_b):
    wb = jnp.concatenate(
        [lr_w.reshape(-1), lr_b.reshape(-1),
         jnp.zeros((15,), jnp.float32)])
    y = _lrrank_sc(uid.astype(jnp.int32), iid.astype(jnp.int32),
                   uid_table.T, iid_table.T, wb)
    return y.reshape(_BATCH, 1)
